# baseline (device time: 47791 ns/iter reference)
import jax
import jax.numpy as jnp
from jax import lax
from jax.experimental import pallas as pl
from jax.experimental.pallas import tpu as pltpu


def kernel(Q, K, V):
    b, sq, h, d = Q.shape
    sk = K.shape[1]
    scale = d ** -0.5

    def body(q_ref, k_ref, v_ref, sc_ref, o_ref,
             k8_ref, v8_ref, rk8_ref, rv8_ref, rsc_ref,
             send_k, send_v, recv_k, recv_v, send_sc, recv_sc):
        my_x = lax.axis_index("x")
        my_y = lax.axis_index("y")
        my_z = lax.axis_index("z")
        partner = (1 - my_x, my_y, my_z)

        barrier_sem = pltpu.get_barrier_semaphore()
        pl.semaphore_signal(
            barrier_sem, inc=1, device_id=partner,
            device_id_type=pl.DeviceIdType.MESH,
        )
        pl.semaphore_wait(barrier_sem, 1)

        rdma_sc = pltpu.make_async_remote_copy(
            src_ref=sc_ref, dst_ref=rsc_ref,
            send_sem=send_sc, recv_sem=recv_sc,
            device_id=partner, device_id_type=pl.DeviceIdType.MESH,
        )
        rdma_sc.start()

        def quantize(xt, s_row):
            xq = xt.astype(jnp.float32) / s_row.astype(jnp.float32)
            return jnp.clip(jnp.round(xq), -127, 127).astype(jnp.int8)

        rdmas = []
        for hi in range(h):
            k8_ref[hi] = quantize(k_ref[hi], sc_ref[0, hi])
            rk = pltpu.make_async_remote_copy(
                src_ref=k8_ref.at[hi], dst_ref=rk8_ref.at[hi],
                send_sem=send_k.at[hi], recv_sem=recv_k.at[hi],
                device_id=partner, device_id_type=pl.DeviceIdType.MESH,
            )
            rk.start()
            v8_ref[hi] = quantize(v_ref[hi], sc_ref[1, hi])
            rv = pltpu.make_async_remote_copy(
                src_ref=v8_ref.at[hi], dst_ref=rv8_ref.at[hi],
                send_sem=send_v.at[hi], recv_sem=recv_v.at[hi],
                device_id=partner, device_id_type=pl.DeviceIdType.MESH,
            )
            rv.start()
            rdmas.append((rk, rv))
        rdma_sc.wait_recv()

        dn_qk = (((2,), (1,)), ((0,), (0,)))
        dn_pv = (((2,), (2,)), ((0,), (0,)))
        for hi in range(h):
            rk, rv = rdmas[hi]
            rk.wait_recv()
            rv.wait_recv()
            q = q_ref[:, :, hi, :].astype(jnp.bfloat16)
            k_loc = k_ref[hi]
            v_loc = v_ref[hi]
            ks_row = rsc_ref[0, hi].astype(jnp.float32)
            vs_row = rsc_ref[1, hi].astype(jnp.float32)

            s_loc = lax.dot_general(
                q, k_loc, dn_qk, preferred_element_type=jnp.float32) * scale
            k_rem = rk8_ref[hi].astype(jnp.bfloat16)
            s_rem = lax.dot_general(
                q, k_rem, dn_qk, preferred_element_type=jnp.float32
            ) * (ks_row * scale)
            m = jnp.maximum(
                jnp.max(s_loc, axis=-1, keepdims=True),
                jnp.max(s_rem, axis=-1, keepdims=True),
            )
            p_loc = jnp.exp(s_loc - m)
            p_rem = jnp.exp(s_rem - m)
            l = (jnp.sum(p_loc, axis=-1, keepdims=True)
                 + jnp.sum(p_rem, axis=-1, keepdims=True))
            o = lax.dot_general(
                p_loc.astype(jnp.bfloat16), v_loc, dn_pv,
                preferred_element_type=jnp.float32)
            v_rem = rv8_ref[hi].astype(jnp.bfloat16)
            o = o + lax.dot_general(
                (p_rem * vs_row).astype(jnp.bfloat16), v_rem, dn_pv,
                preferred_element_type=jnp.float32)
            o_ref[:, :, hi, :] = o / l

        rdma_sc.wait_send()
        for rk, rv in rdmas:
            rk.wait_send()
            rv.wait_send()

    kt = jnp.transpose(K, (2, 0, 3, 1)).astype(jnp.bfloat16)
    vt = jnp.transpose(V, (2, 0, 3, 1)).astype(jnp.bfloat16)

    ks = jnp.max(jnp.abs(kt.astype(jnp.float32)), axis=(1, 2, 3)) / 127.0
    vs = jnp.max(jnp.abs(vt.astype(jnp.float32)), axis=(1, 2, 3)) / 127.0
    sc = jnp.broadcast_to(
        jnp.stack([ks, vs], 0)[:, :, None], (2, h, sk)
    ).astype(jnp.bfloat16)

    return pl.pallas_call(
        body,
        out_shape=jax.ShapeDtypeStruct((b, sq, h, d), jnp.float32),
        in_specs=[pl.BlockSpec(memory_space=pltpu.VMEM)] * 4,
        out_specs=pl.BlockSpec(memory_space=pltpu.VMEM),
        scratch_shapes=[
            pltpu.VMEM((h, b, d, sk), jnp.int8),
            pltpu.VMEM((h, b, d, sk), jnp.int8),
            pltpu.VMEM((h, b, d, sk), jnp.int8),
            pltpu.VMEM((h, b, d, sk), jnp.int8),
            pltpu.VMEM((2, h, sk), jnp.bfloat16),
            pltpu.SemaphoreType.DMA((h,)),
            pltpu.SemaphoreType.DMA((h,)),
            pltpu.SemaphoreType.DMA((h,)),
            pltpu.SemaphoreType.DMA((h,)),
            pltpu.SemaphoreType.DMA,
            pltpu.SemaphoreType.DMA,
        ],
        compiler_params=pltpu.CompilerParams(collective_id=0),
    )(Q, kt, vt, sc)


# device time: 39623 ns/iter; 1.2061x vs baseline; 1.2061x over previous
import jax
import jax.numpy as jnp
from jax import lax
from jax.experimental import pallas as pl
from jax.experimental.pallas import tpu as pltpu


def kernel(Q, K, V):
    b, sq, h, d = Q.shape
    sk = K.shape[1]
    scale = d ** -0.5

    def body(q_ref, k_ref, v_ref, k8_ref, v8_ref, sc_ref, o_ref,
             rk8_ref, rv8_ref, rsc_ref,
             send_k, send_v, recv_k, recv_v, send_sc, recv_sc):
        my_x = lax.axis_index("x")
        my_y = lax.axis_index("y")
        my_z = lax.axis_index("z")
        partner = (1 - my_x, my_y, my_z)

        barrier_sem = pltpu.get_barrier_semaphore()
        pl.semaphore_signal(
            barrier_sem, inc=1, device_id=partner,
            device_id_type=pl.DeviceIdType.MESH,
        )
        pl.semaphore_wait(barrier_sem, 1)

        rdma_sc = pltpu.make_async_remote_copy(
            src_ref=sc_ref, dst_ref=rsc_ref,
            send_sem=send_sc, recv_sem=recv_sc,
            device_id=partner, device_id_type=pl.DeviceIdType.MESH,
        )
        rdma_sc.start()
        rdmas = []
        for hi in range(h):
            rk = pltpu.make_async_remote_copy(
                src_ref=k8_ref.at[hi], dst_ref=rk8_ref.at[hi],
                send_sem=send_k.at[hi], recv_sem=recv_k.at[hi],
                device_id=partner, device_id_type=pl.DeviceIdType.MESH,
            )
            rv = pltpu.make_async_remote_copy(
                src_ref=v8_ref.at[hi], dst_ref=rv8_ref.at[hi],
                send_sem=send_v.at[hi], recv_sem=recv_v.at[hi],
                device_id=partner, device_id_type=pl.DeviceIdType.MESH,
            )
            rk.start()
            rv.start()
            rdmas.append((rk, rv))
        rdma_sc.wait_recv()

        dn_qk = (((2,), (1,)), ((0,), (0,)))
        dn_pv = (((2,), (2,)), ((0,), (0,)))
        for hi in range(h):
            rk, rv = rdmas[hi]
            rk.wait_recv()
            rv.wait_recv()
            q = q_ref[hi]
            k_loc = k_ref[hi]
            v_loc = v_ref[hi]
            ks_row = rsc_ref[0, hi].astype(jnp.float32)
            vs_row = rsc_ref[1, hi].astype(jnp.float32)

            s_loc = lax.dot_general(
                q, k_loc, dn_qk, preferred_element_type=jnp.float32) * scale
            k_rem = rk8_ref[hi].astype(jnp.bfloat16)
            s_rem = lax.dot_general(
                q, k_rem, dn_qk, preferred_element_type=jnp.float32
            ) * (ks_row * scale)
            m = jnp.maximum(
                jnp.max(s_loc, axis=-1, keepdims=True),
                jnp.max(s_rem, axis=-1, keepdims=True),
            )
            p_loc = jnp.exp(s_loc - m)
            p_rem = jnp.exp(s_rem - m)
            l = (jnp.sum(p_loc, axis=-1, keepdims=True)
                 + jnp.sum(p_rem, axis=-1, keepdims=True))
            o = lax.dot_general(
                p_loc.astype(jnp.bfloat16), v_loc, dn_pv,
                preferred_element_type=jnp.float32)
            v_rem = rv8_ref[hi].astype(jnp.bfloat16)
            o = o + lax.dot_general(
                (p_rem * vs_row).astype(jnp.bfloat16), v_rem, dn_pv,
                preferred_element_type=jnp.float32)
            o_ref[hi] = (o / l).astype(jnp.bfloat16)

        rdma_sc.wait_send()
        for rk, rv in rdmas:
            rk.wait_send()
            rv.wait_send()

    qh = jnp.transpose(Q, (2, 0, 1, 3)).astype(jnp.bfloat16)
    kt = jnp.transpose(K, (2, 0, 3, 1)).astype(jnp.bfloat16)
    vt = jnp.transpose(V, (2, 0, 3, 1)).astype(jnp.bfloat16)

    def quant(xt):
        amax = jnp.max(jnp.abs(xt.astype(jnp.float32)), axis=(1, 2, 3))
        s = (amax / 127.0).astype(jnp.bfloat16).astype(jnp.float32)
        x8 = jnp.clip(
            jnp.round(xt.astype(jnp.float32) / s[:, None, None, None]),
            -127, 127).astype(jnp.int8)
        return x8, s
    k8, ks = quant(kt)
    v8, vs = quant(vt)
    sc = jnp.broadcast_to(
        jnp.stack([ks, vs], 0)[:, :, None], (2, h, sk)
    ).astype(jnp.bfloat16)

    out = pl.pallas_call(
        body,
        out_shape=jax.ShapeDtypeStruct((h, b, sq, d), jnp.bfloat16),
        in_specs=[pl.BlockSpec(memory_space=pltpu.VMEM)] * 6,
        out_specs=pl.BlockSpec(memory_space=pltpu.VMEM),
        scratch_shapes=[
            pltpu.VMEM((h, b, d, sk), jnp.int8),
            pltpu.VMEM((h, b, d, sk), jnp.int8),
            pltpu.VMEM((2, h, sk), jnp.bfloat16),
            pltpu.SemaphoreType.DMA((h,)),
            pltpu.SemaphoreType.DMA((h,)),
            pltpu.SemaphoreType.DMA((h,)),
            pltpu.SemaphoreType.DMA((h,)),
            pltpu.SemaphoreType.DMA,
            pltpu.SemaphoreType.DMA,
        ],
        compiler_params=pltpu.CompilerParams(collective_id=0),
    )(qh, kt, vt, k8, v8, sc)
    return jnp.transpose(out, (1, 2, 0, 3))


# device time: 38067 ns/iter; 1.2554x vs baseline; 1.0409x over previous
import jax
import jax.numpy as jnp
from jax import lax
from jax.experimental import pallas as pl
from jax.experimental.pallas import tpu as pltpu


def kernel(Q, K, V):
    b, sq, h, d = Q.shape
    sk = K.shape[1]
    scale = d ** -0.5

    def body(q_ref, k_ref, v_ref, sc_ref, o_ref,
             k8_ref, v8_ref, rk8_ref, rv8_ref, rsc_ref,
             send_k, send_v, recv_k, recv_v, send_sc, recv_sc):
        my_x = lax.axis_index("x")
        my_y = lax.axis_index("y")
        my_z = lax.axis_index("z")
        partner = (1 - my_x, my_y, my_z)

        barrier_sem = pltpu.get_barrier_semaphore()
        pl.semaphore_signal(
            barrier_sem, inc=1, device_id=partner,
            device_id_type=pl.DeviceIdType.MESH,
        )
        pl.semaphore_wait(barrier_sem, 1)

        rdma_sc = pltpu.make_async_remote_copy(
            src_ref=sc_ref, dst_ref=rsc_ref,
            send_sem=send_sc, recv_sem=recv_sc,
            device_id=partner, device_id_type=pl.DeviceIdType.MESH,
        )
        rdma_sc.start()

        def quantize(xt, s_row):
            xq = xt.astype(jnp.float32) / s_row.astype(jnp.float32)
            return jnp.clip(jnp.round(xq), -127, 127).astype(jnp.int8)

        rdmas = []
        for hi in range(h):
            k8_ref[hi] = quantize(k_ref[hi], sc_ref[0, hi])
            rk = pltpu.make_async_remote_copy(
                src_ref=k8_ref.at[hi], dst_ref=rk8_ref.at[hi],
                send_sem=send_k.at[hi], recv_sem=recv_k.at[hi],
                device_id=partner, device_id_type=pl.DeviceIdType.MESH,
            )
            rk.start()
            v8_ref[hi] = quantize(v_ref[hi], sc_ref[1, hi])
            rv = pltpu.make_async_remote_copy(
                src_ref=v8_ref.at[hi], dst_ref=rv8_ref.at[hi],
                send_sem=send_v.at[hi], recv_sem=recv_v.at[hi],
                device_id=partner, device_id_type=pl.DeviceIdType.MESH,
            )
            rv.start()
            rdmas.append((rk, rv))
        rdma_sc.wait_recv()

        dn_qk = (((2,), (1,)), ((0,), (0,)))
        dn_pv = (((2,), (2,)), ((0,), (0,)))
        for hi in range(h):
            rk, rv = rdmas[hi]
            rk.wait_recv()
            rv.wait_recv()
            q = q_ref[hi]
            k_loc = k_ref[hi]
            v_loc = v_ref[hi]
            ks_row = rsc_ref[0, hi].astype(jnp.float32)
            vs_row = rsc_ref[1, hi].astype(jnp.float32)

            s_loc = lax.dot_general(
                q, k_loc, dn_qk, preferred_element_type=jnp.float32) * scale
            k_rem = rk8_ref[hi].astype(jnp.bfloat16)
            s_rem = lax.dot_general(
                q, k_rem, dn_qk, preferred_element_type=jnp.float32
            ) * (ks_row * scale)
            m = jnp.maximum(
                jnp.max(s_loc, axis=-1, keepdims=True),
                jnp.max(s_rem, axis=-1, keepdims=True),
            )
            p_loc = jnp.exp(s_loc - m)
            p_rem = jnp.exp(s_rem - m)
            l = (jnp.sum(p_loc, axis=-1, keepdims=True)
                 + jnp.sum(p_rem, axis=-1, keepdims=True))
            o = lax.dot_general(
                p_loc.astype(jnp.bfloat16), v_loc, dn_pv,
                preferred_element_type=jnp.float32)
            v_rem = rv8_ref[hi].astype(jnp.bfloat16)
            o = o + lax.dot_general(
                (p_rem * vs_row).astype(jnp.bfloat16), v_rem, dn_pv,
                preferred_element_type=jnp.float32)
            o_ref[hi] = (o / l).astype(jnp.bfloat16)

        rdma_sc.wait_send()
        for rk, rv in rdmas:
            rk.wait_send()
            rv.wait_send()

    qh = jnp.transpose(Q, (2, 0, 1, 3)).astype(jnp.bfloat16)
    kt = jnp.transpose(K, (2, 0, 3, 1)).astype(jnp.bfloat16)
    vt = jnp.transpose(V, (2, 0, 3, 1)).astype(jnp.bfloat16)

    ks = jnp.max(jnp.abs(kt.astype(jnp.float32)), axis=(1, 2, 3)) / 127.0
    vs = jnp.max(jnp.abs(vt.astype(jnp.float32)), axis=(1, 2, 3)) / 127.0
    sc = jnp.broadcast_to(
        jnp.stack([ks, vs], 0)[:, :, None], (2, h, sk)
    ).astype(jnp.bfloat16)

    out = pl.pallas_call(
        body,
        out_shape=jax.ShapeDtypeStruct((h, b, sq, d), jnp.bfloat16),
        in_specs=[pl.BlockSpec(memory_space=pltpu.VMEM)] * 4,
        out_specs=pl.BlockSpec(memory_space=pltpu.VMEM),
        scratch_shapes=[
            pltpu.VMEM((h, b, d, sk), jnp.int8),
            pltpu.VMEM((h, b, d, sk), jnp.int8),
            pltpu.VMEM((h, b, d, sk), jnp.int8),
            pltpu.VMEM((h, b, d, sk), jnp.int8),
            pltpu.VMEM((2, h, sk), jnp.bfloat16),
            pltpu.SemaphoreType.DMA((h,)),
            pltpu.SemaphoreType.DMA((h,)),
            pltpu.SemaphoreType.DMA((h,)),
            pltpu.SemaphoreType.DMA((h,)),
            pltpu.SemaphoreType.DMA,
            pltpu.SemaphoreType.DMA,
        ],
        compiler_params=pltpu.CompilerParams(collective_id=0),
    )(qh, kt, vt, sc)
    return jnp.transpose(out, (1, 2, 0, 3))
